# Initial kernel scaffold; baseline (speedup 1.0000x reference)
import jax
import jax.numpy as jnp
from jax.experimental import pallas as pl


def kernel(x, edge_index, batch, W1, b1, W2, b2, W3, b3, Wd1, bd1, Wd2, bd2):
    raise NotImplementedError("write your pallas kernel here")


# speedup vs baseline: 10.6721x; 10.6721x over previous
import functools

import jax
import jax.numpy as jnp
from jax import lax
from jax.experimental import pallas as pl
from jax.experimental.pallas import tpu as pltpu
from jax.experimental.pallas import tpu_sc as plsc

N_NODES = 10000
P_NODES = 10240
D = 128
G = 64
N_EDGES = 320000
NC = 2
NS = 16
NW = NC * NS
CHUNK = 128
CHUNKS_PER_TILE = 79
E_PAD = NW * CHUNKS_PER_TILE * CHUNK
ROWS_PER_TILE = P_NODES // NS
BLK = 2048
HI = lax.Precision.HIGHEST

def _mesh():
    return plsc.VectorSubcoreMesh(core_axis_name="c", subcore_axis_name="s")


def _sc_deg(dst3, zeros16, ones16):
    @functools.partial(
        pl.kernel,
        out_type=jax.ShapeDtypeStruct((NC, P_NODES, 16), jnp.float32),
        mesh=_mesh(),
        scratch_types=[
            pltpu.VMEM((CHUNKS_PER_TILE, CHUNK), jnp.int32),
            pltpu.VMEM((CHUNK, 16), jnp.float32),
            pltpu.VMEM_SHARED((P_NODES, 16), jnp.float32),
        ],
    )
    def deg_kernel(dst_hbm, zeros_hbm, ones_hbm, out_hbm, didx, ones_v, acc):
        c = lax.axis_index("c")
        s = lax.axis_index("s")
        wid = s * NC + c
        r0 = s * ROWS_PER_TILE
        pltpu.sync_copy(zeros_hbm.at[pl.ds(r0, ROWS_PER_TILE)],
                        acc.at[pl.ds(r0, ROWS_PER_TILE)])
        pltpu.sync_copy(dst_hbm.at[wid], didx)
        pltpu.sync_copy(ones_hbm, ones_v)
        plsc.subcore_barrier()

        @pl.loop(0, CHUNKS_PER_TILE)
        def _(j):
            pltpu.sync_copy(ones_v, acc.at[didx.at[j]], add=True)

        plsc.subcore_barrier()
        pltpu.sync_copy(acc.at[pl.ds(r0, ROWS_PER_TILE)],
                        out_hbm.at[c, pl.ds(r0, ROWS_PER_TILE)])

    return deg_kernel(dst3, zeros16, ones16)


def _sc_scatter(g, src3, dst3, zerosP):
    @functools.partial(
        pl.kernel,
        out_type=jax.ShapeDtypeStruct((NC, P_NODES, D), jnp.float32),
        mesh=_mesh(),
        scratch_types=[
            pltpu.VMEM((CHUNKS_PER_TILE, CHUNK), jnp.int32),
            pltpu.VMEM((CHUNKS_PER_TILE, CHUNK), jnp.int32),
            pltpu.VMEM((CHUNK, D), jnp.float32),
            pltpu.VMEM_SHARED((P_NODES, D), jnp.float32),
            pltpu.SemaphoreType.DMA,
        ],
    )
    def scatter_kernel(g_hbm, src_hbm, dst_hbm, zeros_hbm, out_hbm,
                       sidx, didx, rows, acc, sem):
        c = lax.axis_index("c")
        s = lax.axis_index("s")
        wid = s * NC + c
        r0 = s * ROWS_PER_TILE
        pltpu.sync_copy(zeros_hbm.at[pl.ds(r0, ROWS_PER_TILE)],
                        acc.at[pl.ds(r0, ROWS_PER_TILE)])
        pltpu.sync_copy(src_hbm.at[wid], sidx)
        pltpu.sync_copy(dst_hbm.at[wid], didx)
        plsc.subcore_barrier()

        @pl.loop(0, CHUNKS_PER_TILE)
        def _(j):
            pltpu.async_copy(g_hbm.at[sidx.at[j]], rows, sem).wait()
            pltpu.sync_copy(rows, acc.at[didx.at[j]], add=True)

        plsc.subcore_barrier()
        pltpu.sync_copy(acc.at[pl.ds(r0, ROWS_PER_TILE)],
                        out_hbm.at[c, pl.ds(r0, ROWS_PER_TILE)])

    return scatter_kernel(g, src3, dst3, zerosP)


def _dinv(dp_ref):
    deg = 1.0 + dp_ref[0, :, 0:1] + dp_ref[1, :, 0:1]
    return lax.rsqrt(deg)


def _tc_first(xp, W1, degp):
    def body(x_ref, w_ref, dp_ref, o_ref):
        o_ref[...] = _dinv(dp_ref) * jnp.dot(
            x_ref[...], w_ref[...], preferred_element_type=jnp.float32,
            precision=HI)

    return pl.pallas_call(
        body,
        grid=(P_NODES // BLK,),
        in_specs=[
            pl.BlockSpec((BLK, D), lambda i: (i, 0)),
            pl.BlockSpec((D, D), lambda i: (0, 0)),
            pl.BlockSpec((NC, BLK, 16), lambda i: (0, i, 0)),
        ],
        out_specs=pl.BlockSpec((BLK, D), lambda i: (i, 0)),
        out_shape=jax.ShapeDtypeStruct((P_NODES, D), jnp.float32),
    )(xp, W1, degp)


def _tc_mid(pp, g_prev, degp, b, W):
    def body(p_ref, g_ref, dp_ref, b_ref, w_ref, o_ref):
        dinv = _dinv(dp_ref)
        pre = dinv * (p_ref[0] + p_ref[1] + g_ref[...]) + b_ref[...]
        a = jnp.maximum(pre, 0.0)
        o_ref[...] = dinv * jnp.dot(
            a, w_ref[...], preferred_element_type=jnp.float32, precision=HI)

    return pl.pallas_call(
        body,
        grid=(P_NODES // BLK,),
        in_specs=[
            pl.BlockSpec((NC, BLK, D), lambda i: (0, i, 0)),
            pl.BlockSpec((BLK, D), lambda i: (i, 0)),
            pl.BlockSpec((NC, BLK, 16), lambda i: (0, i, 0)),
            pl.BlockSpec((1, D), lambda i: (0, 0)),
            pl.BlockSpec((D, D), lambda i: (0, 0)),
        ],
        out_specs=pl.BlockSpec((BLK, D), lambda i: (i, 0)),
        out_shape=jax.ShapeDtypeStruct((P_NODES, D), jnp.float32),
    )(pp, g_prev, degp, b, W)


def _tc_final(pp, g3, degp, b3, batch_p, Wd1, bd1, Wd2, bd2):
    nblk = P_NODES // BLK

    def body(p_ref, g_ref, dp_ref, b_ref, batch_ref, wd1_ref, bd1_ref,
             wd2_ref, bd2_ref, o_ref, seg, cnt):
        i = pl.program_id(0)

        @pl.when(i == 0)
        def _():
            seg[...] = jnp.zeros_like(seg)
            cnt[...] = jnp.zeros_like(cnt)

        dinv = _dinv(dp_ref)
        out3 = dinv * (p_ref[0] + p_ref[1] + g_ref[...]) + b_ref[...]
        iota = lax.broadcasted_iota(jnp.int32, (BLK, G), 1)
        m = (batch_ref[...] == iota).astype(jnp.float32)
        seg[...] += lax.dot_general(
            m, out3, (((0,), (0,)), ((), ())),
            preferred_element_type=jnp.float32, precision=HI)
        cnt[...] += lax.dot_general(
            m, jnp.ones((BLK, D), jnp.float32), (((0,), (0,)), ((), ())),
            preferred_element_type=jnp.float32, precision=HI)

        @pl.when(i == nblk - 1)
        def _():
            pooled = seg[...] / jnp.maximum(cnt[...], 1.0)
            z = jnp.maximum(
                jnp.dot(pooled, wd1_ref[...],
                        preferred_element_type=jnp.float32, precision=HI)
                + bd1_ref[...], 0.0)
            o_ref[...] = jnp.dot(
                z, wd2_ref[...], preferred_element_type=jnp.float32,
                precision=HI) + bd2_ref[...]

    return pl.pallas_call(
        body,
        grid=(nblk,),
        in_specs=[
            pl.BlockSpec((NC, BLK, D), lambda i: (0, i, 0)),
            pl.BlockSpec((BLK, D), lambda i: (i, 0)),
            pl.BlockSpec((NC, BLK, 16), lambda i: (0, i, 0)),
            pl.BlockSpec((1, D), lambda i: (0, 0)),
            pl.BlockSpec((BLK, 1), lambda i: (i, 0)),
            pl.BlockSpec((D, G), lambda i: (0, 0)),
            pl.BlockSpec((1, G), lambda i: (0, 0)),
            pl.BlockSpec((G, 1), lambda i: (0, 0)),
            pl.BlockSpec((1, 1), lambda i: (0, 0)),
        ],
        out_specs=pl.BlockSpec((G, 1), lambda i: (0, 0)),
        out_shape=jax.ShapeDtypeStruct((G, 1), jnp.float32),
        scratch_shapes=[
            pltpu.VMEM((G, D), jnp.float32),
            pltpu.VMEM((G, D), jnp.float32),
        ],
    )(pp, g3, degp, b3, batch_p, Wd1, bd1, Wd2, bd2)


def kernel(x, edge_index, batch, W1, b1, W2, b2, W3, b3, Wd1, bd1, Wd2, bd2):
    src = edge_index[0].astype(jnp.int32)
    dst = edge_index[1].astype(jnp.int32)
    npad = E_PAD - N_EDGES
    padidx = jnp.full((npad,), P_NODES - 1, jnp.int32)
    src3 = jnp.concatenate([src, padidx]).reshape(NW, CHUNKS_PER_TILE, CHUNK)
    dst3 = jnp.concatenate([dst, padidx]).reshape(NW, CHUNKS_PER_TILE, CHUNK)
    xp = jnp.pad(x, ((0, P_NODES - N_NODES), (0, 0)))
    batch_p = jnp.pad(batch.astype(jnp.int32), (0, P_NODES - N_NODES),
                      constant_values=G).reshape(P_NODES, 1)
    zeros16 = jnp.zeros((P_NODES, 16), jnp.float32)
    ones16 = jnp.ones((CHUNK, 16), jnp.float32)
    zerosP = jnp.zeros((P_NODES, D), jnp.float32)

    degp = _sc_deg(dst3, zeros16, ones16)
    g1 = _tc_first(xp, W1, degp)
    p1 = _sc_scatter(g1, src3, dst3, zerosP)
    g2 = _tc_mid(p1, g1, degp, b1.reshape(1, D), W2)
    p2 = _sc_scatter(g2, src3, dst3, zerosP)
    g3 = _tc_mid(p2, g2, degp, b2.reshape(1, D), W3)
    p3 = _sc_scatter(g3, src3, dst3, zerosP)
    return _tc_final(p3, g3, degp, b3.reshape(1, D), batch_p, Wd1,
                     bd1.reshape(1, G), Wd2, bd2.reshape(1, 1))


# speedup vs baseline: 23.4465x; 2.1970x over previous
import functools

import jax
import jax.numpy as jnp
from jax import lax
from jax.experimental import pallas as pl
from jax.experimental.pallas import tpu as pltpu
from jax.experimental.pallas import tpu_sc as plsc

N_NODES = 10000
P_NODES = 10240
D = 128
G = 64
N_EDGES = 320000
NC = 2
NS = 16
NW = NC * NS
CHUNK = 128
CHUNKS_PER_TILE = 80
NBUF = 2
E_PAD = NW * CHUNKS_PER_TILE * CHUNK
ROWS_PER_TILE = P_NODES // NS
BLK = 2048
HI = lax.Precision.DEFAULT

def _mesh():
    return plsc.VectorSubcoreMesh(core_axis_name="c", subcore_axis_name="s")


def _sc_deg(dst3, zerosP, ones128):
    @functools.partial(
        pl.kernel,
        out_type=jax.ShapeDtypeStruct((NC, P_NODES, D), jnp.float32),
        mesh=_mesh(),
        scratch_types=[
            pltpu.VMEM((CHUNKS_PER_TILE, CHUNK), jnp.int32),
            pltpu.VMEM((CHUNK, D), jnp.float32),
            pltpu.VMEM_SHARED((P_NODES, D), jnp.float32),
            pltpu.SemaphoreType.DMA,
        ],
    )
    def deg_kernel(dst_hbm, zeros_hbm, ones_hbm, out_hbm, didx, ones_v, acc,
                   sem):
        c = lax.axis_index("c")
        s = lax.axis_index("s")
        wid = s * NC + c
        r0 = s * ROWS_PER_TILE
        pltpu.sync_copy(zeros_hbm.at[pl.ds(r0, ROWS_PER_TILE)],
                        acc.at[pl.ds(r0, ROWS_PER_TILE)])
        pltpu.sync_copy(dst_hbm.at[wid], didx)
        pltpu.sync_copy(ones_hbm, ones_v)
        plsc.subcore_barrier()

        wv = 4

        @pl.loop(0, CHUNKS_PER_TILE, step=wv)
        def _(j0):
            for b in range(wv):
                pltpu.async_copy(ones_v, acc.at[didx.at[j0 + b]], sem,
                                 add=True)
            for b in range(wv):
                pltpu.make_async_copy(ones_v, acc.at[didx.at[j0 + b]],
                                      sem).wait()

        plsc.subcore_barrier()
        pltpu.sync_copy(acc.at[pl.ds(r0, ROWS_PER_TILE)],
                        out_hbm.at[c, pl.ds(r0, ROWS_PER_TILE)])

    return deg_kernel(dst3, zerosP, ones128)


def _sc_scatter(g, edges3, zerosP):
    CP = CHUNKS_PER_TILE

    @functools.partial(
        pl.kernel,
        out_type=jax.ShapeDtypeStruct((NC, P_NODES, D), jnp.float32),
        mesh=_mesh(),
        scratch_types=[
            pltpu.VMEM((2 * NBUF, CHUNK), jnp.int32),
            pltpu.VMEM((NBUF, CHUNK, D), jnp.float32),
            pltpu.VMEM_SHARED((P_NODES, D), jnp.float32),
            [pltpu.SemaphoreType.DMA] * NBUF,
            [pltpu.SemaphoreType.DMA] * NBUF,
        ],
    )
    def scatter_kernel(g_hbm, e_hbm, zeros_hbm, out_hbm,
                       idxb, rows, acc, isem, gsem):
        c = lax.axis_index("c")
        s = lax.axis_index("s")
        wid = s * NC + c
        r0 = s * ROWS_PER_TILE

        def idx_start(j, b):
            pltpu.async_copy(e_hbm.at[wid, j], idxb.at[pl.ds(2 * b, 2)],
                             isem[b])

        def idx_wait(j, b):
            pltpu.make_async_copy(e_hbm.at[wid, j], idxb.at[pl.ds(2 * b, 2)],
                                  isem[b]).wait()

        def gat_start(j, b):
            del j
            pltpu.async_copy(g_hbm.at[idxb.at[2 * b]], rows.at[b], gsem[b])

        def gat_wait(j, b):
            del j
            pltpu.make_async_copy(g_hbm.at[idxb.at[2 * b]], rows.at[b],
                                  gsem[b]).wait()

        idx_start(0, 0)
        idx_start(1, 1)
        pltpu.sync_copy(zeros_hbm.at[pl.ds(r0, ROWS_PER_TILE)],
                        acc.at[pl.ds(r0, ROWS_PER_TILE)])
        plsc.subcore_barrier()
        idx_wait(0, 0)
        gat_start(0, 0)

        @pl.loop(0, CP, step=NBUF)
        def _(j0):
            for b in range(NBUF):
                j = j0 + b
                nb = (b + 1) % NBUF

                @pl.when(j + 1 < CP)
                def _():
                    idx_wait(j + 1, nb)
                    gat_start(j + 1, nb)

                gat_wait(j, b)
                pltpu.sync_copy(rows.at[b], acc.at[idxb.at[2 * b + 1]],
                                add=True)

                @pl.when(j + NBUF < CP)
                def _():
                    idx_start(j + NBUF, b)

        plsc.subcore_barrier()
        pltpu.sync_copy(acc.at[pl.ds(r0, ROWS_PER_TILE)],
                        out_hbm.at[c, pl.ds(r0, ROWS_PER_TILE)])

    return scatter_kernel(g, edges3, zerosP)


def _dinv(dp_ref):
    deg = 1.0 + dp_ref[0, :, 0:1] + dp_ref[1, :, 0:1]
    return lax.rsqrt(deg)


def _tc_first(xp, W1, degp):
    def body(x_ref, w_ref, dp_ref, o_ref):
        o_ref[...] = _dinv(dp_ref) * jnp.dot(
            x_ref[...], w_ref[...], preferred_element_type=jnp.float32,
            precision=HI)

    return pl.pallas_call(
        body,
        grid=(P_NODES // BLK,),
        in_specs=[
            pl.BlockSpec((BLK, D), lambda i: (i, 0)),
            pl.BlockSpec((D, D), lambda i: (0, 0)),
            pl.BlockSpec((NC, BLK, D), lambda i: (0, i, 0)),
        ],
        out_specs=pl.BlockSpec((BLK, D), lambda i: (i, 0)),
        out_shape=jax.ShapeDtypeStruct((P_NODES, D), jnp.float32),
    )(xp, W1, degp)


def _tc_mid(pp, g_prev, degp, b, W):
    def body(p_ref, g_ref, dp_ref, b_ref, w_ref, o_ref):
        dinv = _dinv(dp_ref)
        pre = dinv * (p_ref[0] + p_ref[1] + g_ref[...]) + b_ref[...]
        a = jnp.maximum(pre, 0.0)
        o_ref[...] = dinv * jnp.dot(
            a, w_ref[...], preferred_element_type=jnp.float32, precision=HI)

    return pl.pallas_call(
        body,
        grid=(P_NODES // BLK,),
        in_specs=[
            pl.BlockSpec((NC, BLK, D), lambda i: (0, i, 0)),
            pl.BlockSpec((BLK, D), lambda i: (i, 0)),
            pl.BlockSpec((NC, BLK, D), lambda i: (0, i, 0)),
            pl.BlockSpec((1, D), lambda i: (0, 0)),
            pl.BlockSpec((D, D), lambda i: (0, 0)),
        ],
        out_specs=pl.BlockSpec((BLK, D), lambda i: (i, 0)),
        out_shape=jax.ShapeDtypeStruct((P_NODES, D), jnp.float32),
    )(pp, g_prev, degp, b, W)


def _tc_final(pp, g3, degp, b3, batch_p, Wd1, bd1, Wd2, bd2):
    nblk = P_NODES // BLK

    def body(p_ref, g_ref, dp_ref, b_ref, batch_ref, wd1_ref, bd1_ref,
             wd2_ref, bd2_ref, o_ref, seg, cnt):
        i = pl.program_id(0)

        @pl.when(i == 0)
        def _():
            seg[...] = jnp.zeros_like(seg)
            cnt[...] = jnp.zeros_like(cnt)

        dinv = _dinv(dp_ref)
        out3 = dinv * (p_ref[0] + p_ref[1] + g_ref[...]) + b_ref[...]
        iota = lax.broadcasted_iota(jnp.int32, (BLK, G), 1)
        m = (batch_ref[...] == iota).astype(jnp.float32)
        seg[...] += lax.dot_general(
            m, out3, (((0,), (0,)), ((), ())),
            preferred_element_type=jnp.float32, precision=HI)
        cnt[...] += lax.dot_general(
            m, jnp.ones((BLK, D), jnp.float32), (((0,), (0,)), ((), ())),
            preferred_element_type=jnp.float32, precision=HI)

        @pl.when(i == nblk - 1)
        def _():
            pooled = seg[...] / jnp.maximum(cnt[...], 1.0)
            z = jnp.maximum(
                jnp.dot(pooled, wd1_ref[...],
                        preferred_element_type=jnp.float32, precision=HI)
                + bd1_ref[...], 0.0)
            o_ref[...] = jnp.dot(
                z, wd2_ref[...], preferred_element_type=jnp.float32,
                precision=HI) + bd2_ref[...]

    return pl.pallas_call(
        body,
        grid=(nblk,),
        in_specs=[
            pl.BlockSpec((NC, BLK, D), lambda i: (0, i, 0)),
            pl.BlockSpec((BLK, D), lambda i: (i, 0)),
            pl.BlockSpec((NC, BLK, D), lambda i: (0, i, 0)),
            pl.BlockSpec((1, D), lambda i: (0, 0)),
            pl.BlockSpec((BLK, 1), lambda i: (i, 0)),
            pl.BlockSpec((D, G), lambda i: (0, 0)),
            pl.BlockSpec((1, G), lambda i: (0, 0)),
            pl.BlockSpec((G, 1), lambda i: (0, 0)),
            pl.BlockSpec((1, 1), lambda i: (0, 0)),
        ],
        out_specs=pl.BlockSpec((G, 1), lambda i: (0, 0)),
        out_shape=jax.ShapeDtypeStruct((G, 1), jnp.float32),
        scratch_shapes=[
            pltpu.VMEM((G, D), jnp.float32),
            pltpu.VMEM((G, D), jnp.float32),
        ],
    )(pp, g3, degp, b3, batch_p, Wd1, bd1, Wd2, bd2)


def kernel(x, edge_index, batch, W1, b1, W2, b2, W3, b3, Wd1, bd1, Wd2, bd2):
    src = edge_index[0].astype(jnp.int32)
    dst = edge_index[1].astype(jnp.int32)
    npad = E_PAD - N_EDGES
    padidx = N_NODES + jnp.arange(npad, dtype=jnp.int32) % (P_NODES - N_NODES)
    src3 = jnp.concatenate([src, padidx]).reshape(NW, CHUNKS_PER_TILE, CHUNK)
    dst3 = jnp.concatenate([dst, padidx]).reshape(NW, CHUNKS_PER_TILE, CHUNK)
    edges3 = jnp.stack([src3, dst3], axis=2)
    xp = jnp.pad(x, ((0, P_NODES - N_NODES), (0, 0)))
    batch_p = jnp.pad(batch.astype(jnp.int32), (0, P_NODES - N_NODES),
                      constant_values=G).reshape(P_NODES, 1)
    ones128 = jnp.ones((CHUNK, D), jnp.float32)
    zerosP = jnp.zeros((P_NODES, D), jnp.float32)

    degp = _sc_deg(dst3, zerosP, ones128)
    g1 = _tc_first(xp, W1, degp)
    p1 = _sc_scatter(g1, edges3, zerosP)
    g2 = _tc_mid(p1, g1, degp, b1.reshape(1, D), W2)
    p2 = _sc_scatter(g2, edges3, zerosP)
    g3 = _tc_mid(p2, g2, degp, b2.reshape(1, D), W3)
    p3 = _sc_scatter(g3, edges3, zerosP)
    return _tc_final(p3, g3, degp, b3.reshape(1, D), batch_p, Wd1,
                     bd1.reshape(1, G), Wd2, bd2.reshape(1, 1))
